# baseline (speedup 1.0000x reference)
import functools

import jax
import jax.numpy as jnp
from jax import lax
from jax.experimental import pallas as pl
from jax.experimental.pallas import tpu as pltpu
from jax.experimental.pallas import tpu_sc as plsc

_CUT0, _CUT1, _CUT2 = 20000, 200000, 1000000
_D0, _D1, _D2 = 1024, 256, 64
_OUT = 1024
_NTOK = 8192
_NW = 32
_TPW = _NTOK // _NW


def _sc_gather(ids, E0, E1, E2):
    mesh = plsc.VectorSubcoreMesh(core_axis_name="c", subcore_axis_name="s")

    @functools.partial(
        pl.kernel,
        mesh=mesh,
        compiler_params=pltpu.CompilerParams(use_tc_tiling_on_sc=False),
        out_type=[
            jax.ShapeDtypeStruct((_NTOK, _D0), jnp.float32),
            jax.ShapeDtypeStruct((_NTOK, _D1), jnp.float32),
            jax.ShapeDtypeStruct((_NTOK, _D2), jnp.float32),
        ],
        scratch_types=[
            pltpu.VMEM((_TPW,), jnp.int32),
            pltpu.VMEM((_TPW,), jnp.int32),
            pltpu.VMEM((_TPW,), jnp.int32),
            pltpu.VMEM((_TPW,), jnp.int32),
            pltpu.VMEM((32, _D0), jnp.float32),
            pltpu.VMEM((128, _D1), jnp.float32),
            pltpu.VMEM((128, _D2), jnp.float32),
            pltpu.SemaphoreType.DMA,
        ],
    )
    def k(ids_hbm, e0_hbm, e1_hbm, e2_hbm, g0_hbm, g1_hbm, g2_hbm,
          ids_v, i0_v, i1_v, i2_v, r0_v, r1_v, r2_v, sem):
        wid = lax.axis_index("s") * 2 + lax.axis_index("c")
        base = wid * _TPW
        pltpu.sync_copy(ids_hbm.at[pl.ds(base, _TPW)], ids_v)
        for i in range(_TPW // 16):
            sl = pl.ds(i * 16, 16)
            v = ids_v[sl]
            i0_v[sl] = jnp.minimum(v, _CUT0 - 1)
            i1_v[sl] = jnp.minimum(jnp.maximum(v - _CUT0, 0), _CUT1 - _CUT0 - 1)
            i2_v[sl] = jnp.minimum(jnp.maximum(v - _CUT1, 0), _CUT2 - _CUT1 - 1)
        for c in range(8):
            pltpu.async_copy(e0_hbm.at[i0_v.at[pl.ds(c * 32, 32)]], r0_v, sem).wait()
            pltpu.sync_copy(r0_v, g0_hbm.at[pl.ds(base + c * 32, 32), :])
        for c in range(2):
            pltpu.async_copy(e1_hbm.at[i1_v.at[pl.ds(c * 128, 128)]], r1_v, sem).wait()
            pltpu.sync_copy(r1_v, g1_hbm.at[pl.ds(base + c * 128, 128), :])
        for c in range(2):
            pltpu.async_copy(e2_hbm.at[i2_v.at[pl.ds(c * 128, 128)]], r2_v, sem).wait()
            pltpu.sync_copy(r2_v, g2_hbm.at[pl.ds(base + c * 128, 128), :])

    return k(ids, E0, E1, E2)


def _tc_combine(ids_col, G0, G1, G2, W0, W1, W2):
    blk = 512
    grid = (_NTOK // blk,)

    def body(ids_ref, g0_ref, g1_ref, g2_ref, w0_ref, w1_ref, w2_ref, o_ref):
        idb = ids_ref[...]
        m0 = (idb < _CUT0).astype(jnp.float32)
        m1 = ((idb >= _CUT0) & (idb < _CUT1)).astype(jnp.float32)
        m2 = (idb >= _CUT1).astype(jnp.float32)
        acc = jnp.dot(g0_ref[...] * m0, w0_ref[...],
                      preferred_element_type=jnp.float32)
        acc += jnp.dot(g1_ref[...] * m1, w1_ref[...],
                       preferred_element_type=jnp.float32)
        acc += jnp.dot(g2_ref[...] * m2, w2_ref[...],
                       preferred_element_type=jnp.float32)
        o_ref[...] = acc

    return pl.pallas_call(
        body,
        grid=grid,
        in_specs=[
            pl.BlockSpec((blk, 1), lambda i: (i, 0)),
            pl.BlockSpec((blk, _D0), lambda i: (i, 0)),
            pl.BlockSpec((blk, _D1), lambda i: (i, 0)),
            pl.BlockSpec((blk, _D2), lambda i: (i, 0)),
            pl.BlockSpec((_D0, _OUT), lambda i: (0, 0)),
            pl.BlockSpec((_D1, _OUT), lambda i: (0, 0)),
            pl.BlockSpec((_D2, _OUT), lambda i: (0, 0)),
        ],
        out_specs=pl.BlockSpec((blk, _OUT), lambda i: (i, 0)),
        out_shape=jax.ShapeDtypeStruct((_NTOK, _OUT), jnp.float32),
    )(ids_col, G0, G1, G2, W0, W1, W2)


def kernel(input, E0, W0, E1, W1, E2, W2):
    shp = input.shape
    ids = input.reshape(-1).astype(jnp.int32)
    G0, G1, G2 = _sc_gather(ids, E0, E1, E2)
    out = _tc_combine(ids.reshape(-1, 1), G0, G1, G2, W0, W1, W2)
    return out.reshape(shp + (_OUT,))


# speedup vs baseline: 1.8019x; 1.8019x over previous
import functools

import jax
import jax.numpy as jnp
from jax import lax
from jax.experimental import pallas as pl
from jax.experimental.pallas import tpu as pltpu
from jax.experimental.pallas import tpu_sc as plsc

_CUT0, _CUT1, _CUT2 = 20000, 200000, 1000000
_D0, _D1, _D2 = 1024, 256, 64
_OUT = 1024
_NTOK = 8192
_NW = 32
_TPW = _NTOK // _NW
_CH = 16
_NCH = _TPW // _CH


def _sc_gather(ids, E0, E1, E2g):
    mesh = plsc.VectorSubcoreMesh(core_axis_name="c", subcore_axis_name="s")

    @functools.partial(
        pl.kernel,
        mesh=mesh,
        out_type=[
            jax.ShapeDtypeStruct((_NTOK, _D0), jnp.float32),
            jax.ShapeDtypeStruct((_NTOK, _D1), jnp.float32),
            jax.ShapeDtypeStruct((_NTOK, 8, _D2), jnp.float32),
        ],
        scratch_types=[
            pltpu.VMEM((_TPW,), jnp.int32),
            pltpu.VMEM((2, _CH, _D0), jnp.float32),
            pltpu.VMEM((2, _CH, _D1), jnp.float32),
            pltpu.VMEM((2, _CH, 8, _D2), jnp.float32),
            pltpu.SemaphoreType.DMA,
            pltpu.SemaphoreType.DMA,
        ],
    )
    def k(ids_hbm, e0_hbm, e1_hbm, e2g_hbm, g0_hbm, g1_hbm, g2g_hbm,
          ids_v, r0_v, r1_v, r2_v, sem_g, sem_w):
        wid = lax.axis_index("s") * 2 + lax.axis_index("c")
        base = wid * _TPW
        pltpu.sync_copy(ids_hbm.at[pl.ds(base, _TPW)], ids_v)
        def fire_gathers(c, p):
            v = ids_v[pl.ds(c * _CH, _CH)]
            i0 = jnp.minimum(v, _CUT0 - 1)
            i1 = jnp.minimum(jnp.maximum(v - _CUT0, 0), _CUT1 - _CUT0 - 1)
            g2 = jnp.minimum(jnp.maximum(v - _CUT1, 0), _CUT2 - _CUT1 - 1) >> 3
            cps = [pltpu.async_copy(e0_hbm.at[i0], r0_v.at[p], sem_g),
                   pltpu.async_copy(e1_hbm.at[i1], r1_v.at[p], sem_g)]
            for t in range(_CH):
                gt = lax.squeeze(lax.slice(g2, (t,), (t + 1,)), (0,))
                cps.append(pltpu.async_copy(e2g_hbm.at[gt], r2_v.at[p, t], sem_g))
            return cps

        def fire_writebacks(c, p):
            s = base + c * _CH
            w0 = pltpu.async_copy(r0_v.at[p], g0_hbm.at[pl.ds(s, _CH)], sem_w)
            w1 = pltpu.async_copy(r1_v.at[p], g1_hbm.at[pl.ds(s, _CH)], sem_w)
            w2 = pltpu.async_copy(r2_v.at[p], g2g_hbm.at[pl.ds(s, _CH)], sem_w)
            return (w0, w1, w2)

        gs = fire_gathers(0, 0)
        wprev = None
        for c in range(_NCH):
            p = c % 2
            for g in gs:
                g.wait()
            if c + 1 < _NCH:
                if wprev is not None:
                    for w in wprev:
                        w.wait()
                gs = fire_gathers(c + 1, 1 - p)
            wprev = fire_writebacks(c, p)
        for w in wprev:
            w.wait()

    return k(ids, E0, E1, E2g)


def _tc_combine(ids_col, G0, G1, G2g, W0, W1, W2):
    blk = 512
    grid = (_NTOK // blk,)

    def body(ids_ref, g0_ref, g1_ref, g2_ref, w0_ref, w1_ref, w2_ref, o_ref):
        idb = ids_ref[...]
        m0 = (idb < _CUT0).astype(jnp.float32)
        m1 = ((idb >= _CUT0) & (idb < _CUT1)).astype(jnp.float32)
        m2 = (idb >= _CUT1).astype(jnp.float32)
        l2 = jnp.minimum(jnp.maximum(idb - _CUT1, 0), _CUT2 - _CUT1 - 1)
        r = l2 & 7
        g2 = g2_ref[:, 0, :] * (r == 0).astype(jnp.float32)
        for j in range(1, 8):
            g2 += g2_ref[:, j, :] * (r == j).astype(jnp.float32)
        acc = jnp.dot(g0_ref[...] * m0, w0_ref[...],
                      preferred_element_type=jnp.float32)
        acc += jnp.dot(g1_ref[...] * m1, w1_ref[...],
                       preferred_element_type=jnp.float32)
        acc += jnp.dot(g2 * m2, w2_ref[...],
                       preferred_element_type=jnp.float32)
        o_ref[...] = acc

    return pl.pallas_call(
        body,
        grid=grid,
        in_specs=[
            pl.BlockSpec((blk, 1), lambda i: (i, 0)),
            pl.BlockSpec((blk, _D0), lambda i: (i, 0)),
            pl.BlockSpec((blk, _D1), lambda i: (i, 0)),
            pl.BlockSpec((blk, 8, _D2), lambda i: (i, 0, 0)),
            pl.BlockSpec((_D0, _OUT), lambda i: (0, 0)),
            pl.BlockSpec((_D1, _OUT), lambda i: (0, 0)),
            pl.BlockSpec((_D2, _OUT), lambda i: (0, 0)),
        ],
        out_specs=pl.BlockSpec((blk, _OUT), lambda i: (i, 0)),
        out_shape=jax.ShapeDtypeStruct((_NTOK, _OUT), jnp.float32),
    )(ids_col, G0, G1, G2g, W0, W1, W2)


def kernel(input, E0, W0, E1, W1, E2, W2):
    shp = input.shape
    ids = input.reshape(-1).astype(jnp.int32)
    E2g = E2.reshape(100000, 8, _D2)
    G0, G1, G2g = _sc_gather(ids, E0, E1, E2g)
    out = _tc_combine(ids.reshape(-1, 1), G0, G1, G2g, W0, W1, W2)
    return out.reshape(shp + (_OUT,))
